# baseline (speedup 1.0000x reference)
import functools

import jax
import jax.numpy as jnp
from jax import lax
from jax.experimental import pallas as pl
from jax.experimental.pallas import tpu as pltpu
from jax.experimental.pallas import tpu_sc as plsc


_ROWS, _COLS = 16384, 128
_NW = 32
_RPW = _ROWS // _NW
_NCH = 4
_CH = _RPW // _NCH


def _sc_copy(x_hbm, o_hbm, bufs, sem_in, sem_out):
    nc = lax.axis_size("c")
    wid = lax.axis_index("s") * nc + lax.axis_index("c")
    base = wid * _RPW

    ins = []
    for i in range(_NCH):
        cp = pltpu.make_async_copy(
            x_hbm.at[pl.ds(base + i * _CH, _CH), :],
            bufs.at[i], sem_in.at[i])
        cp.start()
        ins.append(cp)

    outs = []
    for i in range(_NCH):
        ins[i].wait()
        if i == 0:
            @pl.when(wid == 0)
            def _():
                lane = lax.iota(jnp.int32, 16)
                head = bufs[0, 0, pl.ds(0, 16)]
                bufs[0, 0, pl.ds(0, 16)] = jnp.where(
                    lane == 0, jnp.float32(3.0), head)
        cp = pltpu.make_async_copy(
            bufs.at[i], o_hbm.at[pl.ds(base + i * _CH, _CH), :],
            sem_out.at[i])
        cp.start()
        outs.append(cp)

    for cp in outs:
        cp.wait()


def kernel(x):
    mesh = plsc.VectorSubcoreMesh(core_axis_name="c", subcore_axis_name="s")
    run = functools.partial(
        pl.kernel,
        mesh=mesh,
        out_type=jax.ShapeDtypeStruct((_ROWS, _COLS), jnp.float32),
        scratch_types=[
            pltpu.VMEM((_NCH, _CH, _COLS), jnp.float32),
            pltpu.SemaphoreType.DMA((_NCH,)),
            pltpu.SemaphoreType.DMA((_NCH,)),
        ],
    )(_sc_copy)
    return run(x)


# speedup vs baseline: 4.4519x; 4.4519x over previous
import jax
import jax.numpy as jnp
from jax.experimental import pallas as pl
from jax.experimental.pallas import tpu as pltpu


_ROWS, _COLS = 16384, 128
_CHUNKS = (512, 1536, 3584, 4096, 3840, 1792, 768, 256)
assert sum(_CHUNKS) == _ROWS
_OFFS = tuple(sum(_CHUNKS[:i]) for i in range(len(_CHUNKS)))
_N = len(_CHUNKS)


def _copy_kernel(x_hbm, o_hbm, buf, sem_in, sem_out):
    ins = []
    for i in range(_N):
        cp = pltpu.make_async_copy(
            x_hbm.at[pl.ds(_OFFS[i], _CHUNKS[i]), :],
            buf.at[pl.ds(_OFFS[i], _CHUNKS[i]), :],
            sem_in.at[i],
        )
        cp.start()
        ins.append(cp)

    outs = []
    for i in range(_N):
        ins[i].wait()
        if i == 0:
            lane = jax.lax.iota(jnp.int32, _COLS)
            head = buf[0, :]
            buf[0, :] = jnp.where(lane == 0, jnp.float32(3.0), head)
        cp = pltpu.make_async_copy(
            buf.at[pl.ds(_OFFS[i], _CHUNKS[i]), :],
            o_hbm.at[pl.ds(_OFFS[i], _CHUNKS[i]), :],
            sem_out.at[i],
        )
        cp.start()
        outs.append(cp)

    for cp in outs:
        cp.wait()


def kernel(x):
    return pl.pallas_call(
        _copy_kernel,
        in_specs=[pl.BlockSpec(memory_space=pl.ANY)],
        out_specs=pl.BlockSpec(memory_space=pl.ANY),
        out_shape=jax.ShapeDtypeStruct((_ROWS, _COLS), x.dtype),
        scratch_shapes=[
            pltpu.VMEM((_ROWS, _COLS), jnp.float32),
            pltpu.SemaphoreType.DMA((_N,)),
            pltpu.SemaphoreType.DMA((_N,)),
        ],
    )(x)
